# baseline (speedup 1.0000x reference)
import functools

import jax
import jax.numpy as jnp
from jax import lax
from jax.experimental import pallas as pl
from jax.experimental.pallas import tpu as pltpu
from jax.experimental.pallas import tpu_sc as plsc

TOKEN_NUM = 1000000
EMB_DIM = 64
B = 1024
X_MAX = 100.0
ALPHA = 0.75

NC = 2
NS = 16
L = 16
NW = NC * NS
B_PER_W = B // NW
LANE_T = 128
NSLOT_R = 2
FB = NSLOT_R


@functools.partial(
    pl.kernel,
    mesh=plsc.VectorSubcoreMesh(core_axis_name="c", subcore_axis_name="s"),
    out_type=jax.ShapeDtypeStruct((B,), jnp.float32),
    compiler_params=pltpu.CompilerParams(needs_layout_passes=False),
    scratch_types=[
        pltpu.VMEM((B_PER_W,), jnp.int32),
        pltpu.VMEM((B_PER_W,), jnp.int32),
        pltpu.VMEM((NSLOT_R + 1, EMB_DIM, LANE_T), jnp.float32),
        pltpu.VMEM((NSLOT_R + 1, EMB_DIM, LANE_T), jnp.float32),
        pltpu.VMEM((NSLOT_R + 1, LANE_T), jnp.float32),
        pltpu.VMEM((NSLOT_R + 1, LANE_T), jnp.float32),
        pltpu.VMEM((L * L,), jnp.float32),
        pltpu.VMEM((B_PER_W,), jnp.float32),
        pltpu.SemaphoreType.DMA,
    ],
)
def _sc_pred(xt_hbm, emb_it_hbm, emb_jt_hbm, bit_hbm, bjt_hbm,
             pred_hbm, idx_iv, idx_jv, slab_i, slab_j, bslab_i, bslab_j,
             tmp_v, pred_v, sem):
    wid = lax.axis_index("s") * NC + lax.axis_index("c")
    base = wid * B_PER_W
    icp = pltpu.make_async_copy(xt_hbm.at[0, pl.ds(base, B_PER_W)],
                                idx_iv, sem)
    jcp = pltpu.make_async_copy(xt_hbm.at[1, pl.ds(base, B_PER_W)],
                                idx_jv, sem)
    icp.start()
    jcp.start()
    icp.wait()
    jcp.wait()

    ivecs = [idx_iv[pl.ds(g * L, L)] for g in range(B_PER_W // L)]
    jvecs = [idx_jv[pl.ds(g * L, L)] for g in range(B_PER_W // L)]

    tis = [v >> 7 for v in ivecs]
    tjs = [v >> 7 for v in jvecs]
    last_base = jnp.int32((TOKEN_NUM - 1) // LANE_T - (NSLOT_R - 1))
    base_i = jnp.minimum(jnp.minimum(jnp.min(tis[0]), jnp.min(tis[1])),
                         last_base)
    base_j = jnp.minimum(jnp.minimum(jnp.min(tjs[0]), jnp.min(tjs[1])),
                         last_base)
    range_cps = []
    for s in range(NSLOT_R):
        ib = pl.multiple_of((base_i + s) << 7, LANE_T)
        jb = pl.multiple_of((base_j + s) << 7, LANE_T)
        range_cps.append(pltpu.async_copy(
            emb_it_hbm.at[:, pl.ds(ib, LANE_T)], slab_i.at[s], sem))
        range_cps.append(pltpu.async_copy(
            emb_jt_hbm.at[:, pl.ds(jb, LANE_T)], slab_j.at[s], sem))
        range_cps.append(pltpu.async_copy(
            bit_hbm.at[0, pl.ds(ib, LANE_T)], bslab_i.at[s], sem))
        range_cps.append(pltpu.async_copy(
            bjt_hbm.at[0, pl.ds(jb, LANE_T)], bslab_j.at[s], sem))
    for cp in range_cps:
        cp.wait()

    lane_idx = lax.iota(jnp.int32, L)
    for g in range(B_PER_W // L):
        iv, jv = ivecs[g], jvecs[g]
        off_vi = tis[g] - base_i
        off_vj = tjs[g] - base_j
        inr_vi = off_vi < NSLOT_R
        inr_vj = off_vj < NSLOT_R
        bias_fix_i = jnp.zeros((L,), jnp.float32)
        bias_fix_j = jnp.zeros((L,), jnp.float32)
        for p in range(L):
            off_i = off_vi[p]
            off_j = off_vj[p]
            inr_i = off_i < NSLOT_R
            inr_j = off_j < NSLOT_R
            li = jnp.broadcast_to(iv[p] & (LANE_T - 1), (L,))
            lj = jnp.broadcast_to(jv[p] & (LANE_T - 1), (L,))

            @pl.when(jnp.logical_not(inr_i))
            def _(idx=iv[p]):
                ib = pl.multiple_of((idx >> 7) << 7, LANE_T)
                pltpu.sync_copy(emb_it_hbm.at[:, pl.ds(ib, LANE_T)],
                                slab_i.at[FB])
                pltpu.sync_copy(bit_hbm.at[0, pl.ds(ib, LANE_T)],
                                bslab_i.at[FB])

            @pl.when(jnp.logical_not(inr_j))
            def _(idx=jv[p]):
                jb = pl.multiple_of((idx >> 7) << 7, LANE_T)
                pltpu.sync_copy(emb_jt_hbm.at[:, pl.ds(jb, LANE_T)],
                                slab_j.at[FB])
                pltpu.sync_copy(bjt_hbm.at[0, pl.ds(jb, LANE_T)],
                                bslab_j.at[FB])

            take_i = jnp.logical_and(lane_idx == p,
                                     jnp.broadcast_to(~inr_i, (L,)))
            take_j = jnp.logical_and(lane_idx == p,
                                     jnp.broadcast_to(~inr_j, (L,)))
            fbs = jnp.full((L,), FB, jnp.int32)
            bias_fix_i = jnp.where(
                take_i, plsc.load_gather(bslab_i, [fbs, li]), bias_fix_i)
            bias_fix_j = jnp.where(
                take_j, plsc.load_gather(bslab_j, [fbs, lj]), bias_fix_j)

            sl_i = jnp.broadcast_to(
                jnp.where(inr_i, off_i, jnp.int32(FB)), (L,))
            sl_j = jnp.broadcast_to(
                jnp.where(inr_j, off_j, jnp.int32(FB)), (L,))
            prod = None
            for d0 in range(0, EMB_DIM, L):
                dvec = lane_idx + d0
                a = plsc.load_gather(slab_i, [sl_i, dvec, li])
                bb = plsc.load_gather(slab_j, [sl_j, dvec, lj])
                prod = a * bb if prod is None else prod + a * bb
            tmp_v[pl.ds(p * L, L)] = prod

        slot_vi = jnp.where(inr_vi, off_vi, FB)
        slot_vj = jnp.where(inr_vj, off_vj, FB)
        bias_i = plsc.load_gather(bslab_i, [slot_vi, iv & (LANE_T - 1)])
        bias_j = plsc.load_gather(bslab_j, [slot_vj, jv & (LANE_T - 1)])
        acc = (jnp.where(inr_vi, bias_i, bias_fix_i)
               + jnp.where(inr_vj, bias_j, bias_fix_j))
        rows = lane_idx * L
        for d in range(L):
            acc = acc + plsc.load_gather(tmp_v, [rows + d])
        pred_v[pl.ds(g * L, L)] = acc

    pltpu.sync_copy(pred_v, pred_hbm.at[pl.ds(base, B_PER_W)])


_ROW_BLKS = 16
_ROWS = B // _ROW_BLKS


def _tc_outer_body(pred_ref, xij_ref, out_ref):
    xf = xij_ref[:, :].astype(jnp.float32)
    logx = jnp.log(xf)
    fx = jnp.where(xf >= X_MAX, jnp.float32(1.0),
                   jnp.exp(ALPHA * jnp.log(xf / X_MAX)))
    diff = pred_ref[:, :] - logx
    out_ref[:, 0, :] = fx * diff * diff


_tc_outer = pl.pallas_call(
    _tc_outer_body,
    grid=(_ROW_BLKS,),
    in_specs=[
        pl.BlockSpec((_ROWS, 1), lambda i: (i, 0)),
        pl.BlockSpec((1, B), lambda i: (0, 0)),
    ],
    out_specs=pl.BlockSpec((_ROWS, 1, B), lambda i: (i, 0, 0)),
    out_shape=jax.ShapeDtypeStruct((B, 1, B), jnp.float32),
)


def kernel(x, emb_i, emb_j, bi, bj):
    xij = x[:, 2]
    pred = _sc_pred(x.T, emb_i.T, emb_j.T, bi.T, bj.T)
    return _tc_outer(pred.reshape(B, 1), xij.reshape(1, B))


# speedup vs baseline: 1.1149x; 1.1149x over previous
import functools

import jax
import jax.numpy as jnp
from jax import lax
from jax.experimental import pallas as pl
from jax.experimental.pallas import tpu as pltpu
from jax.experimental.pallas import tpu_sc as plsc

TOKEN_NUM = 1000000
EMB_DIM = 64
B = 1024
X_MAX = 100.0
ALPHA = 0.75

NC = 2
NS = 16
L = 16
NW = NC * NS
B_PER_W = B // NW
LANE_T = 128
NSLOT_R = 2
FB = NSLOT_R


@functools.partial(
    pl.kernel,
    mesh=plsc.VectorSubcoreMesh(core_axis_name="c", subcore_axis_name="s"),
    out_type=jax.ShapeDtypeStruct((B,), jnp.float32),
    compiler_params=pltpu.CompilerParams(needs_layout_passes=False),
    scratch_types=[
        pltpu.VMEM((B_PER_W,), jnp.int32),
        pltpu.VMEM((B_PER_W,), jnp.int32),
        pltpu.VMEM((NSLOT_R + 1, EMB_DIM, LANE_T), jnp.float32),
        pltpu.VMEM((NSLOT_R + 1, EMB_DIM, LANE_T), jnp.float32),
        pltpu.VMEM((NSLOT_R + 1, LANE_T), jnp.float32),
        pltpu.VMEM((NSLOT_R + 1, LANE_T), jnp.float32),
        pltpu.VMEM((L * L,), jnp.float32),
        pltpu.VMEM((B_PER_W,), jnp.float32),
        pltpu.SemaphoreType.DMA,
    ],
)
def _sc_pred(xt_hbm, emb_it_hbm, emb_jt_hbm, bit_hbm, bjt_hbm,
             pred_hbm, idx_iv, idx_jv, slab_i, slab_j, bslab_i, bslab_j,
             tmp_v, pred_v, sem):
    wid = lax.axis_index("s") * NC + lax.axis_index("c")
    base = wid * B_PER_W
    icp = pltpu.make_async_copy(xt_hbm.at[0, pl.ds(base, B_PER_W)],
                                idx_iv, sem)
    jcp = pltpu.make_async_copy(xt_hbm.at[1, pl.ds(base, B_PER_W)],
                                idx_jv, sem)
    icp.start()
    jcp.start()
    icp.wait()
    jcp.wait()

    ivecs = [idx_iv[pl.ds(g * L, L)] for g in range(B_PER_W // L)]
    jvecs = [idx_jv[pl.ds(g * L, L)] for g in range(B_PER_W // L)]

    tis = [v >> 7 for v in ivecs]
    tjs = [v >> 7 for v in jvecs]
    last_base = jnp.int32((TOKEN_NUM - 1) // LANE_T - (NSLOT_R - 1))
    base_i = jnp.minimum(jnp.minimum(jnp.min(tis[0]), jnp.min(tis[1])),
                         last_base)
    base_j = jnp.minimum(jnp.minimum(jnp.min(tjs[0]), jnp.min(tjs[1])),
                         last_base)
    range_cps = []
    for s in range(NSLOT_R):
        ib = pl.multiple_of((base_i + s) << 7, LANE_T)
        jb = pl.multiple_of((base_j + s) << 7, LANE_T)
        range_cps.append(pltpu.async_copy(
            emb_it_hbm.at[:, pl.ds(ib, LANE_T)], slab_i.at[s], sem))
        range_cps.append(pltpu.async_copy(
            emb_jt_hbm.at[:, pl.ds(jb, LANE_T)], slab_j.at[s], sem))
        range_cps.append(pltpu.async_copy(
            bit_hbm.at[0, pl.ds(ib, LANE_T)], bslab_i.at[s], sem))
        range_cps.append(pltpu.async_copy(
            bjt_hbm.at[0, pl.ds(jb, LANE_T)], bslab_j.at[s], sem))
    for cp in range_cps:
        cp.wait()

    lane_idx = lax.iota(jnp.int32, L)
    for g in range(B_PER_W // L):
        iv, jv = ivecs[g], jvecs[g]
        off_vi = tis[g] - base_i
        off_vj = tjs[g] - base_j
        inr_vi = off_vi < NSLOT_R
        inr_vj = off_vj < NSLOT_R
        bias_fix_i = jnp.zeros((L,), jnp.float32)
        bias_fix_j = jnp.zeros((L,), jnp.float32)
        for p in range(L):
            off_i = off_vi[p]
            off_j = off_vj[p]
            inr_i = off_i < NSLOT_R
            inr_j = off_j < NSLOT_R
            li = jnp.broadcast_to(iv[p] & (LANE_T - 1), (L,))
            lj = jnp.broadcast_to(jv[p] & (LANE_T - 1), (L,))

            @pl.when(jnp.logical_not(inr_i))
            def _(idx=iv[p]):
                ib = pl.multiple_of((idx >> 7) << 7, LANE_T)
                pltpu.sync_copy(emb_it_hbm.at[:, pl.ds(ib, LANE_T)],
                                slab_i.at[FB])
                pltpu.sync_copy(bit_hbm.at[0, pl.ds(ib, LANE_T)],
                                bslab_i.at[FB])

            @pl.when(jnp.logical_not(inr_j))
            def _(idx=jv[p]):
                jb = pl.multiple_of((idx >> 7) << 7, LANE_T)
                pltpu.sync_copy(emb_jt_hbm.at[:, pl.ds(jb, LANE_T)],
                                slab_j.at[FB])
                pltpu.sync_copy(bjt_hbm.at[0, pl.ds(jb, LANE_T)],
                                bslab_j.at[FB])

            take_i = jnp.logical_and(lane_idx == p,
                                     jnp.broadcast_to(~inr_i, (L,)))
            take_j = jnp.logical_and(lane_idx == p,
                                     jnp.broadcast_to(~inr_j, (L,)))
            fbs = jnp.full((L,), FB, jnp.int32)
            bias_fix_i = jnp.where(
                take_i, plsc.load_gather(bslab_i, [fbs, li]), bias_fix_i)
            bias_fix_j = jnp.where(
                take_j, plsc.load_gather(bslab_j, [fbs, lj]), bias_fix_j)

            sl_i = jnp.broadcast_to(
                jnp.where(inr_i, off_i, jnp.int32(FB)), (L,))
            sl_j = jnp.broadcast_to(
                jnp.where(inr_j, off_j, jnp.int32(FB)), (L,))
            prod = None
            for d0 in range(0, EMB_DIM, L):
                dvec = lane_idx + d0
                a = plsc.load_gather(slab_i, [sl_i, dvec, li])
                bb = plsc.load_gather(slab_j, [sl_j, dvec, lj])
                prod = a * bb if prod is None else prod + a * bb
            tmp_v[pl.ds(p * L, L)] = prod

        slot_vi = jnp.where(inr_vi, off_vi, FB)
        slot_vj = jnp.where(inr_vj, off_vj, FB)
        bias_i = plsc.load_gather(bslab_i, [slot_vi, iv & (LANE_T - 1)])
        bias_j = plsc.load_gather(bslab_j, [slot_vj, jv & (LANE_T - 1)])
        acc = (jnp.where(inr_vi, bias_i, bias_fix_i)
               + jnp.where(inr_vj, bias_j, bias_fix_j))
        rows = lane_idx * L
        for d in range(L):
            acc = acc + plsc.load_gather(tmp_v, [rows + d])
        pred_v[pl.ds(g * L, L)] = acc

    pltpu.sync_copy(pred_v, pred_hbm.at[pl.ds(base, B_PER_W)])


_ROW_BLKS = 8
_ROWS = B // _ROW_BLKS


def _tc_outer_body(pred_ref, xij_ref, out_ref):
    xf = xij_ref[:, :].astype(jnp.float32)
    logx = jnp.log(xf)
    fx = jnp.where(xf >= X_MAX, jnp.float32(1.0),
                   jnp.exp(ALPHA * jnp.log(xf / X_MAX)))
    diff = pred_ref[:, :] - logx
    out_ref[:, 0, :] = fx * diff * diff


_tc_outer = pl.pallas_call(
    _tc_outer_body,
    grid=(_ROW_BLKS,),
    in_specs=[
        pl.BlockSpec((_ROWS, 1), lambda i: (i, 0)),
        pl.BlockSpec((1, B), lambda i: (0, 0)),
    ],
    out_specs=pl.BlockSpec((_ROWS, 1, B), lambda i: (i, 0, 0)),
    out_shape=jax.ShapeDtypeStruct((B, 1, B), jnp.float32),
)


def kernel(x, emb_i, emb_j, bi, bj):
    xij = x[:, 2]
    pred = _sc_pred(x.T, emb_i.T, emb_j.T, bi.T, bj.T)
    return _tc_outer(pred.reshape(B, 1), xij.reshape(1, B))


# speedup vs baseline: 1.1436x; 1.0258x over previous
import functools

import jax
import jax.numpy as jnp
from jax import lax
from jax.experimental import pallas as pl
from jax.experimental.pallas import tpu as pltpu
from jax.experimental.pallas import tpu_sc as plsc

TOKEN_NUM = 1000000
EMB_DIM = 64
B = 1024
X_MAX = 100.0
ALPHA = 0.75

NC = 2
NS = 16
L = 16
NW = NC * NS
B_PER_W = B // NW
LANE_T = 128
NSLOT_R = 2
FB = NSLOT_R


ROWS_PER_IT = 4


@functools.partial(
    pl.kernel,
    mesh=plsc.VectorSubcoreMesh(core_axis_name="c", subcore_axis_name="s"),
    out_type=jax.ShapeDtypeStruct((B, 1, B), jnp.float32),
    compiler_params=pltpu.CompilerParams(needs_layout_passes=False),
    scratch_types=[
        pltpu.VMEM((B_PER_W,), jnp.int32),
        pltpu.VMEM((B_PER_W,), jnp.int32),
        pltpu.VMEM((NSLOT_R + 1, EMB_DIM, LANE_T), jnp.float32),
        pltpu.VMEM((NSLOT_R + 1, EMB_DIM, LANE_T), jnp.float32),
        pltpu.VMEM((NSLOT_R + 1, LANE_T), jnp.float32),
        pltpu.VMEM((NSLOT_R + 1, LANE_T), jnp.float32),
        pltpu.VMEM((L * L,), jnp.float32),
        pltpu.VMEM((B_PER_W,), jnp.float32),
        pltpu.VMEM((B,), jnp.float32),
        pltpu.VMEM((B,), jnp.float32),
        pltpu.VMEM((ROWS_PER_IT, 1, B), jnp.float32),
        pltpu.SemaphoreType.DMA,
    ],
)
def _sc_glove(xt_hbm, emb_it_hbm, emb_jt_hbm, bit_hbm, bjt_hbm,
              logx_hbm, fx_hbm, out_hbm, idx_iv, idx_jv, slab_i, slab_j,
              bslab_i, bslab_j, tmp_v, pred_v, logx_v, fx_v, sbuf, sem):
    wid = lax.axis_index("s") * NC + lax.axis_index("c")
    base = wid * B_PER_W
    icp = pltpu.make_async_copy(xt_hbm.at[0, pl.ds(base, B_PER_W)],
                                idx_iv, sem)
    jcp = pltpu.make_async_copy(xt_hbm.at[1, pl.ds(base, B_PER_W)],
                                idx_jv, sem)
    icp.start()
    jcp.start()
    icp.wait()
    jcp.wait()

    ivecs = [idx_iv[pl.ds(g * L, L)] for g in range(B_PER_W // L)]
    jvecs = [idx_jv[pl.ds(g * L, L)] for g in range(B_PER_W // L)]

    tis = [v >> 7 for v in ivecs]
    tjs = [v >> 7 for v in jvecs]
    last_base = jnp.int32((TOKEN_NUM - 1) // LANE_T - (NSLOT_R - 1))
    base_i = jnp.minimum(jnp.minimum(jnp.min(tis[0]), jnp.min(tis[1])),
                         last_base)
    base_j = jnp.minimum(jnp.minimum(jnp.min(tjs[0]), jnp.min(tjs[1])),
                         last_base)
    range_cps = []
    for s in range(NSLOT_R):
        ib = pl.multiple_of((base_i + s) << 7, LANE_T)
        jb = pl.multiple_of((base_j + s) << 7, LANE_T)
        range_cps.append(pltpu.async_copy(
            emb_it_hbm.at[:, pl.ds(ib, LANE_T)], slab_i.at[s], sem))
        range_cps.append(pltpu.async_copy(
            emb_jt_hbm.at[:, pl.ds(jb, LANE_T)], slab_j.at[s], sem))
        range_cps.append(pltpu.async_copy(
            bit_hbm.at[0, pl.ds(ib, LANE_T)], bslab_i.at[s], sem))
        range_cps.append(pltpu.async_copy(
            bjt_hbm.at[0, pl.ds(jb, LANE_T)], bslab_j.at[s], sem))
    range_cps.append(pltpu.async_copy(logx_hbm.at[0], logx_v, sem))
    range_cps.append(pltpu.async_copy(fx_hbm.at[0], fx_v, sem))
    for cp in range_cps:
        cp.wait()

    lane_idx = lax.iota(jnp.int32, L)
    for g in range(B_PER_W // L):
        iv, jv = ivecs[g], jvecs[g]
        off_vi = tis[g] - base_i
        off_vj = tjs[g] - base_j
        inr_vi = off_vi < NSLOT_R
        inr_vj = off_vj < NSLOT_R
        bias_fix_i = jnp.zeros((L,), jnp.float32)
        bias_fix_j = jnp.zeros((L,), jnp.float32)
        for p in range(L):
            off_i = off_vi[p]
            off_j = off_vj[p]
            inr_i = off_i < NSLOT_R
            inr_j = off_j < NSLOT_R
            li = jnp.broadcast_to(iv[p] & (LANE_T - 1), (L,))
            lj = jnp.broadcast_to(jv[p] & (LANE_T - 1), (L,))

            @pl.when(jnp.logical_not(inr_i))
            def _(idx=iv[p]):
                ib = pl.multiple_of((idx >> 7) << 7, LANE_T)
                pltpu.sync_copy(emb_it_hbm.at[:, pl.ds(ib, LANE_T)],
                                slab_i.at[FB])
                pltpu.sync_copy(bit_hbm.at[0, pl.ds(ib, LANE_T)],
                                bslab_i.at[FB])

            @pl.when(jnp.logical_not(inr_j))
            def _(idx=jv[p]):
                jb = pl.multiple_of((idx >> 7) << 7, LANE_T)
                pltpu.sync_copy(emb_jt_hbm.at[:, pl.ds(jb, LANE_T)],
                                slab_j.at[FB])
                pltpu.sync_copy(bjt_hbm.at[0, pl.ds(jb, LANE_T)],
                                bslab_j.at[FB])

            take_i = jnp.logical_and(lane_idx == p,
                                     jnp.broadcast_to(~inr_i, (L,)))
            take_j = jnp.logical_and(lane_idx == p,
                                     jnp.broadcast_to(~inr_j, (L,)))
            fbs = jnp.full((L,), FB, jnp.int32)
            bias_fix_i = jnp.where(
                take_i, plsc.load_gather(bslab_i, [fbs, li]), bias_fix_i)
            bias_fix_j = jnp.where(
                take_j, plsc.load_gather(bslab_j, [fbs, lj]), bias_fix_j)

            sl_i = jnp.broadcast_to(
                jnp.where(inr_i, off_i, jnp.int32(FB)), (L,))
            sl_j = jnp.broadcast_to(
                jnp.where(inr_j, off_j, jnp.int32(FB)), (L,))
            prod = None
            for d0 in range(0, EMB_DIM, L):
                dvec = lane_idx + d0
                a = plsc.load_gather(slab_i, [sl_i, dvec, li])
                bb = plsc.load_gather(slab_j, [sl_j, dvec, lj])
                prod = a * bb if prod is None else prod + a * bb
            tmp_v[pl.ds(p * L, L)] = prod

        slot_vi = jnp.where(inr_vi, off_vi, FB)
        slot_vj = jnp.where(inr_vj, off_vj, FB)
        bias_i = plsc.load_gather(bslab_i, [slot_vi, iv & (LANE_T - 1)])
        bias_j = plsc.load_gather(bslab_j, [slot_vj, jv & (LANE_T - 1)])
        acc = (jnp.where(inr_vi, bias_i, bias_fix_i)
               + jnp.where(inr_vj, bias_j, bias_fix_j))
        rows = lane_idx * L
        for d in range(L):
            acc = acc + plsc.load_gather(tmp_v, [rows + d])
        pred_v[pl.ds(g * L, L)] = acc

    def _rows_body(rg, _):
        for k in range(ROWS_PER_IT):
            r = rg * ROWS_PER_IT + k
            pb = plsc.load_gather(pred_v, [jnp.broadcast_to(r, (L,))])
            for c in range(B // L):
                sl = pl.ds(c * L, L)
                diff = pb - logx_v[sl]
                sbuf[k, 0, sl] = fx_v[sl] * diff * diff
        pltpu.sync_copy(
            sbuf,
            out_hbm.at[pl.ds(base + rg * ROWS_PER_IT, ROWS_PER_IT),
                       pl.ds(0, 1), pl.ds(0, B)])
        return _

    lax.fori_loop(0, B_PER_W // ROWS_PER_IT, _rows_body, None)


def _tc_lf_body(xij_ref, logx_ref, fx_ref):
    xf = xij_ref[:, :].astype(jnp.float32)
    logx_ref[:, :] = jnp.log(xf)
    fx_ref[:, :] = jnp.where(xf >= X_MAX, jnp.float32(1.0),
                             jnp.exp(ALPHA * jnp.log(xf / X_MAX)))


_tc_lf = pl.pallas_call(
    _tc_lf_body,
    out_shape=(jax.ShapeDtypeStruct((1, B), jnp.float32),
               jax.ShapeDtypeStruct((1, B), jnp.float32)),
)


def kernel(x, emb_i, emb_j, bi, bj):
    xij = x[:, 2]
    logx, fx = _tc_lf(xij.reshape(1, B))
    return _sc_glove(x.T, emb_i.T, emb_j.T, bi.T, bj.T, logx, fx)


# speedup vs baseline: 1.2334x; 1.0785x over previous
import functools

import jax
import jax.numpy as jnp
from jax import lax
from jax.experimental import pallas as pl
from jax.experimental.pallas import tpu as pltpu
from jax.experimental.pallas import tpu_sc as plsc

TOKEN_NUM = 1000000
EMB_DIM = 64
B = 1024
X_MAX = 100.0
ALPHA = 0.75

NC = 2
NS = 16
L = 16
NW = NC * NS
B_PER_W = B // NW
LANE_T = 128
NSLOT_R = 2
FB = NSLOT_R


ROWS_PER_IT = 4


@functools.partial(
    pl.kernel,
    mesh=plsc.VectorSubcoreMesh(core_axis_name="c", subcore_axis_name="s"),
    out_type=jax.ShapeDtypeStruct((B, 1, B), jnp.float32),
    compiler_params=pltpu.CompilerParams(needs_layout_passes=False),
    scratch_types=[
        pltpu.VMEM((B_PER_W,), jnp.int32),
        pltpu.VMEM((B_PER_W,), jnp.int32),
        pltpu.VMEM((NSLOT_R + 1, EMB_DIM, LANE_T), jnp.float32),
        pltpu.VMEM((NSLOT_R + 1, EMB_DIM, LANE_T), jnp.float32),
        pltpu.VMEM((NSLOT_R + 1, LANE_T), jnp.float32),
        pltpu.VMEM((NSLOT_R + 1, LANE_T), jnp.float32),
        pltpu.VMEM((L * L,), jnp.float32),
        pltpu.VMEM((B_PER_W,), jnp.float32),
        pltpu.VMEM((B,), jnp.float32),
        pltpu.VMEM((B,), jnp.float32),
        pltpu.VMEM((ROWS_PER_IT, 1, B), jnp.float32),
        pltpu.VMEM((ROWS_PER_IT, 1, B), jnp.float32),
        pltpu.SemaphoreType.DMA,
        pltpu.SemaphoreType.DMA,
        pltpu.SemaphoreType.DMA,
    ],
)
def _sc_glove(xt_hbm, emb_it_hbm, emb_jt_hbm, bit_hbm, bjt_hbm,
              logx_hbm, fx_hbm, out_hbm, idx_iv, idx_jv, slab_i, slab_j,
              bslab_i, bslab_j, tmp_v, pred_v, logx_v, fx_v, sbuf0, sbuf1,
              sem, sem_a, sem_b):
    wid = lax.axis_index("s") * NC + lax.axis_index("c")
    base = wid * B_PER_W
    icp = pltpu.make_async_copy(xt_hbm.at[0, pl.ds(base, B_PER_W)],
                                idx_iv, sem)
    jcp = pltpu.make_async_copy(xt_hbm.at[1, pl.ds(base, B_PER_W)],
                                idx_jv, sem)
    icp.start()
    jcp.start()
    icp.wait()
    jcp.wait()

    ivecs = [idx_iv[pl.ds(g * L, L)] for g in range(B_PER_W // L)]
    jvecs = [idx_jv[pl.ds(g * L, L)] for g in range(B_PER_W // L)]

    tis = [v >> 7 for v in ivecs]
    tjs = [v >> 7 for v in jvecs]
    last_base = jnp.int32((TOKEN_NUM - 1) // LANE_T - (NSLOT_R - 1))
    base_i = jnp.minimum(jnp.minimum(jnp.min(tis[0]), jnp.min(tis[1])),
                         last_base)
    base_j = jnp.minimum(jnp.minimum(jnp.min(tjs[0]), jnp.min(tjs[1])),
                         last_base)
    range_cps = []
    for s in range(NSLOT_R):
        ib = pl.multiple_of((base_i + s) << 7, LANE_T)
        jb = pl.multiple_of((base_j + s) << 7, LANE_T)
        range_cps.append(pltpu.async_copy(
            emb_it_hbm.at[:, pl.ds(ib, LANE_T)], slab_i.at[s], sem))
        range_cps.append(pltpu.async_copy(
            emb_jt_hbm.at[:, pl.ds(jb, LANE_T)], slab_j.at[s], sem))
        range_cps.append(pltpu.async_copy(
            bit_hbm.at[0, pl.ds(ib, LANE_T)], bslab_i.at[s], sem))
        range_cps.append(pltpu.async_copy(
            bjt_hbm.at[0, pl.ds(jb, LANE_T)], bslab_j.at[s], sem))
    range_cps.append(pltpu.async_copy(logx_hbm.at[0], logx_v, sem))
    range_cps.append(pltpu.async_copy(fx_hbm.at[0], fx_v, sem))
    for cp in range_cps:
        cp.wait()

    lane_idx = lax.iota(jnp.int32, L)
    for g in range(B_PER_W // L):
        iv, jv = ivecs[g], jvecs[g]
        off_vi = tis[g] - base_i
        off_vj = tjs[g] - base_j
        inr_vi = off_vi < NSLOT_R
        inr_vj = off_vj < NSLOT_R
        bias_fix_i = jnp.zeros((L,), jnp.float32)
        bias_fix_j = jnp.zeros((L,), jnp.float32)
        for p in range(L):
            off_i = off_vi[p]
            off_j = off_vj[p]
            inr_i = off_i < NSLOT_R
            inr_j = off_j < NSLOT_R
            li = jnp.broadcast_to(iv[p] & (LANE_T - 1), (L,))
            lj = jnp.broadcast_to(jv[p] & (LANE_T - 1), (L,))

            @pl.when(jnp.logical_not(inr_i))
            def _(idx=iv[p]):
                ib = pl.multiple_of((idx >> 7) << 7, LANE_T)
                pltpu.sync_copy(emb_it_hbm.at[:, pl.ds(ib, LANE_T)],
                                slab_i.at[FB])
                pltpu.sync_copy(bit_hbm.at[0, pl.ds(ib, LANE_T)],
                                bslab_i.at[FB])

            @pl.when(jnp.logical_not(inr_j))
            def _(idx=jv[p]):
                jb = pl.multiple_of((idx >> 7) << 7, LANE_T)
                pltpu.sync_copy(emb_jt_hbm.at[:, pl.ds(jb, LANE_T)],
                                slab_j.at[FB])
                pltpu.sync_copy(bjt_hbm.at[0, pl.ds(jb, LANE_T)],
                                bslab_j.at[FB])

            take_i = jnp.logical_and(lane_idx == p,
                                     jnp.broadcast_to(~inr_i, (L,)))
            take_j = jnp.logical_and(lane_idx == p,
                                     jnp.broadcast_to(~inr_j, (L,)))
            fbs = jnp.full((L,), FB, jnp.int32)
            bias_fix_i = jnp.where(
                take_i, plsc.load_gather(bslab_i, [fbs, li]), bias_fix_i)
            bias_fix_j = jnp.where(
                take_j, plsc.load_gather(bslab_j, [fbs, lj]), bias_fix_j)

            sl_i = jnp.broadcast_to(
                jnp.where(inr_i, off_i, jnp.int32(FB)), (L,))
            sl_j = jnp.broadcast_to(
                jnp.where(inr_j, off_j, jnp.int32(FB)), (L,))
            prod = None
            for d0 in range(0, EMB_DIM, L):
                dvec = lane_idx + d0
                a = plsc.load_gather(slab_i, [sl_i, dvec, li])
                bb = plsc.load_gather(slab_j, [sl_j, dvec, lj])
                prod = a * bb if prod is None else prod + a * bb
            tmp_v[pl.ds(p * L, L)] = prod

        slot_vi = jnp.where(inr_vi, off_vi, FB)
        slot_vj = jnp.where(inr_vj, off_vj, FB)
        bias_i = plsc.load_gather(bslab_i, [slot_vi, iv & (LANE_T - 1)])
        bias_j = plsc.load_gather(bslab_j, [slot_vj, jv & (LANE_T - 1)])
        acc = (jnp.where(inr_vi, bias_i, bias_fix_i)
               + jnp.where(inr_vj, bias_j, bias_fix_j))
        rows = lane_idx * L
        for d in range(L):
            acc = acc + plsc.load_gather(tmp_v, [rows + d])
        pred_v[pl.ds(g * L, L)] = acc

    def _out_slice(g0):
        return out_hbm.at[pl.ds(base + g0 * ROWS_PER_IT, ROWS_PER_IT),
                          pl.ds(0, 1), pl.ds(0, B)]

    def _rows_body(rg, _):
        for half, buf, bsem in ((0, sbuf0, sem_a), (1, sbuf1, sem_b)):
            g0 = rg * 2 + half

            @pl.when(rg > 0)
            def _(buf=buf, bsem=bsem, g0=g0):
                pltpu.make_async_copy(buf, _out_slice(g0), bsem).wait()

            pbs = [plsc.load_gather(
                       pred_v,
                       [jnp.broadcast_to(g0 * ROWS_PER_IT + k, (L,))])
                   for k in range(ROWS_PER_IT)]
            for c in range(B // L):
                sl = pl.ds(c * L, L)
                lx = logx_v[sl]
                fxc = fx_v[sl]
                for k in range(ROWS_PER_IT):
                    diff = pbs[k] - lx
                    buf[k, 0, sl] = fxc * diff * diff
            pltpu.make_async_copy(buf, _out_slice(g0), bsem).start()
        return _

    n_it = B_PER_W // ROWS_PER_IT // 2
    lax.fori_loop(0, n_it, _rows_body, None)
    last = 2 * (n_it - 1)
    pltpu.make_async_copy(sbuf0, _out_slice(last), sem_a).wait()
    pltpu.make_async_copy(sbuf1, _out_slice(last + 1), sem_b).wait()


def _tc_lf_body(xij_ref, logx_ref, fx_ref):
    xf = xij_ref[:, :].astype(jnp.float32)
    logx_ref[:, :] = jnp.log(xf)
    fx_ref[:, :] = jnp.where(xf >= X_MAX, jnp.float32(1.0),
                             jnp.exp(ALPHA * jnp.log(xf / X_MAX)))


_tc_lf = pl.pallas_call(
    _tc_lf_body,
    out_shape=(jax.ShapeDtypeStruct((1, B), jnp.float32),
               jax.ShapeDtypeStruct((1, B), jnp.float32)),
)


def kernel(x, emb_i, emb_j, bi, bj):
    xij = x[:, 2]
    logx, fx = _tc_lf(xij.reshape(1, B))
    return _sc_glove(x.T, emb_i.T, emb_j.T, bi.T, bj.T, logx, fx)
